# baseline (speedup 1.0000x reference)
import functools

import jax
import jax.numpy as jnp
from jax.experimental import pallas as pl
from jax.experimental.pallas import tpu as pltpu

_BN = 2048
_SLOTS = 4
_BG = 16


def _sweep_body(t_ref, q_ref, w_ref, train_ref, act_ref,
                u_ref, mb_ref, sb_ref, predb_ref,
                zq_ref, wbuf_ref, sem, *, n_total, nb_total):
    nb = pl.program_id(0)

    @pl.when(nb == 0)
    def _init():
        zq_ref[...] = jnp.dot(q_ref[...], w_ref[...],
                              preferred_element_type=jnp.float32)

    slot = jax.lax.rem(nb, _SLOTS)

    @pl.when(nb >= _SLOTS)
    def _wait_prev():
        pltpu.make_async_copy(
            wbuf_ref.at[slot],
            u_ref.at[:, pl.ds((nb - _SLOTS) * _BN, _BN)],
            sem.at[slot],
        ).wait()

    zt = jnp.dot(train_ref[...], w_ref[...],
                 preferred_element_type=jnp.float32)
    sim = jnp.dot(zq_ref[...], zt.T, preferred_element_type=jnp.float32)
    logits = sim / t_ref[0]
    col = nb * _BN + jax.lax.broadcasted_iota(jnp.int32, logits.shape, 1)
    logits = jnp.where(col < n_total, logits, -1e30)

    m_b = jnp.max(logits, axis=1, keepdims=True)
    u = jnp.exp(logits - m_b)
    mb_ref[0] = m_b
    sb_ref[0] = jnp.sum(u, axis=1, keepdims=True)

    wbuf_ref[slot] = u
    pltpu.make_async_copy(
        wbuf_ref.at[slot],
        u_ref.at[:, pl.ds(nb * _BN, _BN)],
        sem.at[slot],
    ).start()

    arow = jax.lax.broadcasted_iota(jnp.int32, act_ref.shape, 0) + nb * _BN
    act = jnp.where(arow < n_total, act_ref[...], 0.0)
    predb_ref[0] = jnp.dot(u, act, preferred_element_type=jnp.float32)

    @pl.when(nb == nb_total - 1)
    def _drain():
        for j in range(max(0, nb_total - _SLOTS), nb_total):
            s_slot = j % _SLOTS
            pltpu.make_async_copy(
                wbuf_ref.at[s_slot],
                u_ref.at[:, pl.ds(j * _BN, _BN)],
                sem.at[s_slot],
            ).wait()


def _repack_body(c_ref, predb_ref, u_ref, w_ref, pred_ref,
                 ubuf_ref, wstage_ref, insem, outsem, *,
                 n_total, nb_total, bg, g_total):
    g = pl.program_id(0)
    buf = jax.lax.rem(g, 2)

    def _in_copy(j, b):
        return pltpu.make_async_copy(
            u_ref.at[pl.ds(j * bg, bg), :],
            ubuf_ref.at[b],
            insem.at[b],
        )

    def _out_copy(j, b):
        return pltpu.make_async_copy(
            wstage_ref.at[b],
            w_ref.at[pl.ds(j * bg, bg), :],
            outsem.at[b],
        )

    @pl.when(g == 0)
    def _prologue():
        _in_copy(0, 0).start()

    @pl.when(g + 1 < g_total)
    def _prefetch():
        _in_copy(g + 1, jax.lax.rem(g + 1, 2)).start()

    _in_copy(g, buf).wait()

    @pl.when(g >= 2)
    def _wait_out_prev():
        _out_copy(g - 2, buf).wait()

    u = ubuf_ref[buf]
    for b in range(nb_total):
        c_b = c_ref[b]
        lo = b * _BN
        hi = min((b + 1) * _BN, n_total)
        wstage_ref[buf, :, lo:hi] = u[:, lo:hi] * c_b

    _out_copy(g, buf).start()

    acc = jnp.zeros(pred_ref.shape, jnp.float32)
    for b in range(nb_total):
        acc = acc + c_ref[b] * predb_ref[b]
    pred_ref[...] = acc

    @pl.when(g == g_total - 1)
    def _drain():
        _out_copy(g - 1, jax.lax.rem(g - 1, 2)).wait()
        _out_copy(g, buf).wait()


def kernel(query_obs, train_obs, train_actions, W_enc, log_temperature):
    B, d = query_obs.shape
    N = train_obs.shape[0]
    H, A = train_actions.shape[1], train_actions.shape[2]
    HA = H * A
    nb_total = pl.cdiv(N, _BN)
    n_pad = nb_total * _BN

    temp = jnp.exp(log_temperature).reshape(1)
    act_flat = train_actions.reshape(N, HA)

    scalar_spec = pl.BlockSpec(memory_space=pltpu.SMEM)

    u, m_b, s_b, pred_b = pl.pallas_call(
        functools.partial(_sweep_body, n_total=N, nb_total=nb_total),
        grid=(nb_total,),
        in_specs=[
            scalar_spec,
            pl.BlockSpec((B, d), lambda nb: (0, 0)),
            pl.BlockSpec((d, d), lambda nb: (0, 0)),
            pl.BlockSpec((_BN, d), lambda nb: (nb, 0)),
            pl.BlockSpec((_BN, HA), lambda nb: (nb, 0)),
        ],
        out_specs=[
            pl.BlockSpec(memory_space=pltpu.MemorySpace.HBM),
            pl.BlockSpec((1, B, 1), lambda nb: (nb, 0, 0)),
            pl.BlockSpec((1, B, 1), lambda nb: (nb, 0, 0)),
            pl.BlockSpec((1, B, HA), lambda nb: (nb, 0, 0)),
        ],
        out_shape=[
            jax.ShapeDtypeStruct((B, n_pad), jnp.float32),
            jax.ShapeDtypeStruct((nb_total, B, 1), jnp.float32),
            jax.ShapeDtypeStruct((nb_total, B, 1), jnp.float32),
            jax.ShapeDtypeStruct((nb_total, B, HA), jnp.float32),
        ],
        scratch_shapes=[
            pltpu.VMEM((B, d), jnp.float32),
            pltpu.VMEM((_SLOTS, B, _BN), jnp.float32),
            pltpu.SemaphoreType.DMA((_SLOTS,)),
        ],
        compiler_params=pltpu.CompilerParams(
            dimension_semantics=("arbitrary",),
        ),
    )(temp, query_obs, W_enc, train_obs, act_flat)

    m = jnp.max(m_b, axis=0, keepdims=True)
    scale = jnp.exp(m_b - m)
    s = jnp.sum(s_b * scale, axis=0, keepdims=True)
    c = scale / s

    g_total = B // _BG
    weights, pred = pl.pallas_call(
        functools.partial(_repack_body, n_total=N, nb_total=nb_total,
                          bg=_BG, g_total=g_total),
        grid=(g_total,),
        in_specs=[
            pl.BlockSpec((nb_total, _BG, 1), lambda g: (0, g, 0)),
            pl.BlockSpec((nb_total, _BG, HA), lambda g: (0, g, 0)),
            pl.BlockSpec(memory_space=pltpu.MemorySpace.HBM),
        ],
        out_specs=[
            pl.BlockSpec(memory_space=pltpu.MemorySpace.HBM),
            pl.BlockSpec((_BG, HA), lambda g: (g, 0)),
        ],
        out_shape=[
            jax.ShapeDtypeStruct((B, N), jnp.float32),
            jax.ShapeDtypeStruct((B, HA), jnp.float32),
        ],
        scratch_shapes=[
            pltpu.VMEM((2, _BG, n_pad), jnp.float32),
            pltpu.VMEM((2, _BG, N), jnp.float32),
            pltpu.SemaphoreType.DMA((2,)),
            pltpu.SemaphoreType.DMA((2,)),
        ],
        compiler_params=pltpu.CompilerParams(
            dimension_semantics=("arbitrary",),
        ),
    )(c, pred_b, u)

    return (pred.reshape(B, H, A), weights)


# speedup vs baseline: 1.4510x; 1.4510x over previous
import functools

import jax
import jax.numpy as jnp
from jax.experimental import pallas as pl
from jax.experimental.pallas import tpu as pltpu

_BN = 2048
_SLOTS = 4


def _stats_body(t_ref, q_ref, w_ref, train_ref, m_ref, s_ref, zq_ref, *,
                n_total):
    nb = pl.program_id(0)

    @pl.when(nb == 0)
    def _init():
        zq_ref[...] = jnp.dot(q_ref[...], w_ref[...],
                              preferred_element_type=jnp.float32)
        m_ref[...] = jnp.full_like(m_ref, -1e30)
        s_ref[...] = jnp.zeros_like(s_ref)

    zt = jnp.dot(train_ref[...], w_ref[...],
                 preferred_element_type=jnp.float32)
    sim = jnp.dot(zq_ref[...], zt.T, preferred_element_type=jnp.float32)
    logits = sim / t_ref[0]
    col = nb * _BN + jax.lax.broadcasted_iota(jnp.int32, logits.shape, 1)
    logits = jnp.where(col < n_total, logits, -1e30)

    m_old = m_ref[...]
    m_new = jnp.maximum(m_old, jnp.max(logits, axis=1, keepdims=True))
    s_ref[...] = (s_ref[...] * jnp.exp(m_old - m_new)
                  + jnp.sum(jnp.exp(logits - m_new), axis=1, keepdims=True))
    m_ref[...] = m_new


def _combine_body(t_ref, q_ref, w_ref, train_ref, act_ref, m_ref, s_ref,
                  wout_ref, pred_ref, zq_ref, wbuf_ref, sem, *,
                  n_total, nb_total, n_pad_tail):
    nb = pl.program_id(0)

    @pl.when(nb == 0)
    def _init():
        zq_ref[...] = jnp.dot(q_ref[...], w_ref[...],
                              preferred_element_type=jnp.float32)
        pred_ref[...] = jnp.zeros_like(pred_ref)

    slot = jax.lax.rem(nb, _SLOTS)

    @pl.when(jnp.logical_and(nb >= _SLOTS, nb - _SLOTS <= nb_total - 2))
    def _wait_prev():
        pltpu.make_async_copy(
            wbuf_ref.at[slot],
            wout_ref.at[:, pl.ds((nb - _SLOTS) * _BN, _BN)],
            sem.at[slot],
        ).wait()

    zt = jnp.dot(train_ref[...], w_ref[...],
                 preferred_element_type=jnp.float32)
    sim = jnp.dot(zq_ref[...], zt.T, preferred_element_type=jnp.float32)
    logits = sim / t_ref[0]
    col = nb * _BN + jax.lax.broadcasted_iota(jnp.int32, logits.shape, 1)
    w = jnp.exp(logits - m_ref[...]) / s_ref[...]
    w = jnp.where(col < n_total, w, 0.0)

    wbuf_ref[slot] = w

    @pl.when(nb < nb_total - 1)
    def _start_full():
        pltpu.make_async_copy(
            wbuf_ref.at[slot],
            wout_ref.at[:, pl.ds(nb * _BN, _BN)],
            sem.at[slot],
        ).start()

    @pl.when(nb == nb_total - 1)
    def _start_tail():
        pltpu.make_async_copy(
            wbuf_ref.at[slot, :, pl.ds(0, n_pad_tail)],
            wout_ref.at[:, pl.ds(nb * _BN, n_pad_tail)],
            sem.at[slot],
        ).start()

    arow = jax.lax.broadcasted_iota(jnp.int32, act_ref.shape, 0) + nb * _BN
    act = jnp.where(arow < n_total, act_ref[...], 0.0)
    pred_ref[...] += jnp.dot(w, act, preferred_element_type=jnp.float32)

    @pl.when(nb == nb_total - 1)
    def _drain():
        for j in range(max(0, nb_total - _SLOTS), nb_total - 1):
            s_slot = j % _SLOTS
            pltpu.make_async_copy(
                wbuf_ref.at[s_slot],
                wout_ref.at[:, pl.ds(j * _BN, _BN)],
                sem.at[s_slot],
            ).wait()
        pltpu.make_async_copy(
            wbuf_ref.at[slot, :, pl.ds(0, n_pad_tail)],
            wout_ref.at[:, pl.ds(nb * _BN, n_pad_tail)],
            sem.at[slot],
        ).wait()


def _tail_patch_body(t_ref, wmain_ref, q_ref, w_ref, train_ref, m_ref, s_ref,
                     wout_ref, *, n_total, nb_total):
    zq = jnp.dot(q_ref[...], w_ref[...], preferred_element_type=jnp.float32)
    zt = jnp.dot(train_ref[...], w_ref[...],
                 preferred_element_type=jnp.float32)
    sim = jnp.dot(zq, zt.T, preferred_element_type=jnp.float32)
    logits = sim / t_ref[0]
    col = ((nb_total - 1) * _BN
           + jax.lax.broadcasted_iota(jnp.int32, logits.shape, 1))
    w = jnp.exp(logits - m_ref[...]) / s_ref[...]
    wout_ref[...] = jnp.where(col < n_total, w, 0.0)


def kernel(query_obs, train_obs, train_actions, W_enc, log_temperature):
    B, d = query_obs.shape
    N = train_obs.shape[0]
    H, A = train_actions.shape[1], train_actions.shape[2]
    HA = H * A
    nb_total = pl.cdiv(N, _BN)
    n_pad128 = pl.cdiv(N, 128) * 128
    n_pad_tail = n_pad128 - (nb_total - 1) * _BN

    temp = jnp.exp(log_temperature).reshape(1)
    act_flat = train_actions.reshape(N, HA)

    scalar_spec = pl.BlockSpec(memory_space=pltpu.SMEM)

    m, s = pl.pallas_call(
        functools.partial(_stats_body, n_total=N),
        grid=(nb_total,),
        in_specs=[
            scalar_spec,
            pl.BlockSpec((B, d), lambda nb: (0, 0)),
            pl.BlockSpec((d, d), lambda nb: (0, 0)),
            pl.BlockSpec((_BN, d), lambda nb: (nb, 0)),
        ],
        out_specs=[
            pl.BlockSpec((B, 1), lambda nb: (0, 0)),
            pl.BlockSpec((B, 1), lambda nb: (0, 0)),
        ],
        out_shape=[
            jax.ShapeDtypeStruct((B, 1), jnp.float32),
            jax.ShapeDtypeStruct((B, 1), jnp.float32),
        ],
        scratch_shapes=[pltpu.VMEM((B, d), jnp.float32)],
        compiler_params=pltpu.CompilerParams(
            dimension_semantics=("arbitrary",),
        ),
    )(temp, query_obs, W_enc, train_obs)

    w_main, pred = pl.pallas_call(
        functools.partial(_combine_body, n_total=N, nb_total=nb_total,
                          n_pad_tail=n_pad_tail),
        grid=(nb_total,),
        in_specs=[
            scalar_spec,
            pl.BlockSpec((B, d), lambda nb: (0, 0)),
            pl.BlockSpec((d, d), lambda nb: (0, 0)),
            pl.BlockSpec((_BN, d), lambda nb: (nb, 0)),
            pl.BlockSpec((_BN, HA), lambda nb: (nb, 0)),
            pl.BlockSpec((B, 1), lambda nb: (0, 0)),
            pl.BlockSpec((B, 1), lambda nb: (0, 0)),
        ],
        out_specs=[
            pl.BlockSpec(memory_space=pltpu.MemorySpace.HBM),
            pl.BlockSpec((B, HA), lambda nb: (0, 0)),
        ],
        out_shape=[
            jax.ShapeDtypeStruct((B, n_pad128), jnp.float32),
            jax.ShapeDtypeStruct((B, HA), jnp.float32),
        ],
        scratch_shapes=[
            pltpu.VMEM((B, d), jnp.float32),
            pltpu.VMEM((_SLOTS, B, _BN), jnp.float32),
            pltpu.SemaphoreType.DMA((_SLOTS,)),
        ],
        compiler_params=pltpu.CompilerParams(
            dimension_semantics=("arbitrary",),
        ),
    )(temp, query_obs, W_enc, train_obs, act_flat, m, s)

    weights = jax.lax.slice(w_main, (0, 0), (B, N))

    return (pred.reshape(B, H, A), weights)
